# Initial kernel scaffold; baseline (speedup 1.0000x reference)
import jax
import jax.numpy as jnp
from jax.experimental import pallas as pl


def kernel(indices, embedding_matrix, dense_W, dense_b):
    raise NotImplementedError("write your pallas kernel here")


# speedup vs baseline: 3.1832x; 3.1832x over previous
import functools

import jax
import jax.numpy as jnp
from jax import lax
from jax.experimental import pallas as pl
from jax.experimental.pallas import tpu as pltpu
from jax.experimental.pallas import tpu_sc as plsc

NC, NS, LANES = 2, 16, 16
NW = NC * NS
B = 1024
T = 1000
TPAD = 1024
CHUNK = 128
NCHUNK = TPAD // CHUNK
D = 32
BPW = B // NW
ROW_UNROLL = 8


def _sc_body(idx_hbm, table_hbm, w_hbm, b_hbm, out_hbm,
             idx_v, rows_v, feat_v, w_v, bias_v, out_v, sem):
    wid = lax.axis_index("s") * NC + lax.axis_index("c")
    base = wid * BPW
    iota = lax.iota(jnp.int32, LANES)

    pltpu.sync_copy(w_hbm, w_v)
    pltpu.sync_copy(b_hbm, bias_v)

    def batch_body(i, carry):
        b = base + i
        pltpu.sync_copy(idx_hbm.at[b], idx_v)
        copies = [
            pltpu.async_copy(table_hbm.at[idx_v.at[j]],
                             rows_v.at[pl.ds(j * CHUNK, CHUNK)], sem)
            for j in range(NCHUNK)
        ]
        for c in copies:
            c.wait()

        zeros = jnp.zeros((LANES,), jnp.float32)
        init = (zeros, zeros, jnp.float32(-1.0), zeros, zeros)

        def row_body(it, c):
            s0, s1, m, bv0, bv1 = c
            for u in range(ROW_UNROLL):
                r = it * ROW_UNROLL + u
                a = rows_v[r, pl.ds(0, LANES)]
                bb = rows_v[r, pl.ds(LANES, LANES)]
                s0 = s0 + a
                s1 = s1 + bb
                nsq = jnp.sum(a * a + bb * bb)
                pred = nsq > m
                m = jnp.where(pred, nsq, m)
                pv = jnp.broadcast_to(pred, (LANES,))
                bv0 = jnp.where(pv, a, bv0)
                bv1 = jnp.where(pv, bb, bv1)
            return (s0, s1, m, bv0, bv1)

        s0, s1, m, bv0, bv1 = lax.fori_loop(0, T // ROW_UNROLL, row_body, init)
        inv = jnp.float32(1.0 / T)
        off = i * (2 * D)
        feat_v[pl.ds(off, LANES)] = s0 * inv
        feat_v[pl.ds(off + LANES, LANES)] = s1 * inv
        feat_v[pl.ds(off + 2 * LANES, LANES)] = bv0
        feat_v[pl.ds(off + 3 * LANES, LANES)] = bv1
        return carry

    lax.fori_loop(0, BPW, batch_body, 0)

    bvec = bias_v[pl.ds(0, LANES)]
    b0 = bvec[0]
    b1 = bvec[1]
    w0vecs = [w_v[pl.ds(k * LANES, LANES)] for k in range(2 * D // LANES)]
    w1vecs = [w_v[pl.ds(2 * D + k * LANES, LANES)] for k in range(2 * D // LANES)]
    iota_feat = iota * (2 * D)
    for g in range(BPW // LANES):
        acc0 = jnp.broadcast_to(b0, (LANES,))
        acc1 = jnp.broadcast_to(b1, (LANES,))
        gbase = g * LANES * (2 * D)
        for d in range(2 * D):
            v = plsc.load_gather(feat_v, [iota_feat + (gbase + d)])
            acc0 = acc0 + v * w0vecs[d // LANES][d % LANES]
            acc1 = acc1 + v * w1vecs[d // LANES][d % LANES]
        p0 = 1.0 / (1.0 + jnp.exp(-acc0))
        p1 = 1.0 / (1.0 + jnp.exp(-acc1))
        row_idx = g * LANES + iota
        plsc.store_scatter(out_v, [row_idx, jnp.zeros((LANES,), jnp.int32)], p0)
        plsc.store_scatter(out_v, [row_idx, jnp.ones((LANES,), jnp.int32)], p1)

    pltpu.sync_copy(out_v, out_hbm.at[pl.ds(base, BPW)])


@jax.jit
def _logreg_sc(idx3, table, wflat, bpad):
    mesh = plsc.VectorSubcoreMesh(core_axis_name="c", subcore_axis_name="s",
                                  num_cores=NC, num_subcores=NS)
    fn = pl.kernel(
        _sc_body,
        out_type=jax.ShapeDtypeStruct((B, 2), jnp.float32),
        mesh=mesh,
        compiler_params=pltpu.CompilerParams(needs_layout_passes=False,
                                             use_tc_tiling_on_sc=False),
        scratch_types=[
            pltpu.VMEM((NCHUNK, CHUNK), jnp.int32),
            pltpu.VMEM((TPAD, D), jnp.float32),
            pltpu.VMEM((BPW * 2 * D,), jnp.float32),
            pltpu.VMEM((2 * 2 * D,), jnp.float32),
            pltpu.VMEM((LANES,), jnp.float32),
            pltpu.VMEM((BPW, 2), jnp.float32),
            pltpu.SemaphoreType.DMA,
        ],
    )
    return fn(idx3, table, wflat, bpad)


def kernel(indices, embedding_matrix, dense_W, dense_b):
    idx = indices.reshape(B, T).astype(jnp.int32)
    idx3 = jnp.pad(idx, ((0, 0), (0, TPAD - T))).reshape(B, NCHUNK, CHUNK)
    wflat = dense_W.astype(jnp.float32).T.reshape(2 * 2 * D)
    bpad = jnp.pad(dense_b.astype(jnp.float32), (0, LANES - 2))
    return _logreg_sc(idx3, embedding_matrix, wflat, bpad)
